# Initial kernel scaffold; baseline (speedup 1.0000x reference)
import jax
import jax.numpy as jnp
from jax.experimental import pallas as pl


def kernel(h, edge_index, edge_order, W, b):
    raise NotImplementedError("write your pallas kernel here")


# speedup vs baseline: 3.5568x; 3.5568x over previous
import jax
import jax.numpy as jnp
from jax import lax
from jax.experimental import pallas as pl
from jax.experimental.pallas import tpu as pltpu
from jax.experimental.pallas import tpu_sc as plsc

N = 10000
E = 320000
D = 128
DH = D // 2

NS = 16
C = 128
K = (E + NS * C - 1) // (NS * C)
EP = NS * C * K
NPAD = 10112
RPT = NPAD // NS


def _sc_body(h2_hbm, src_hbm, dst_hbm, ord_hbm, agg_out, deg_out,
             src_v, dst_v, ord_v, rows_v, dbuf_v, dzero_v, agg_sh, deg_sh, sem):
    cid = lax.axis_index("c")
    sid = lax.axis_index("s")

    pltpu.sync_copy(src_hbm.at[sid], src_v)
    pltpu.sync_copy(dst_hbm.at[sid], dst_v)
    pltpu.sync_copy(ord_hbm.at[sid], ord_v)

    zeros16 = jnp.zeros((16,), jnp.float32)
    lanes = lax.iota(jnp.int32, 16)
    onevec = jnp.where(lanes == 0, 1.0, 0.0).astype(jnp.float32)
    off = (cid * N).astype(jnp.int32)

    @pl.loop(0, K)
    def _shift_idx(j):
        for q in range(C // 16):
            v = src_v[j, pl.ds(16 * q, 16)]
            src_v[j, pl.ds(16 * q, 16)] = v + off

    @pl.loop(0, C)
    def _zero_bufs(c):
        for q in range(DH // 16):
            rows_v[c, pl.ds(16 * q, 16)] = zeros16
        dbuf_v[c, pl.ds(0, 16)] = onevec
        dzero_v[c, pl.ds(0, 16)] = zeros16

    base = sid * RPT
    for k in range(RPT // C):
        pltpu.sync_copy(rows_v, agg_sh.at[pl.ds(base + C * k, C)])
        pltpu.sync_copy(dzero_v, deg_sh.at[pl.ds(base + C * k, C)])
    rem = RPT % C
    pltpu.sync_copy(rows_v.at[pl.ds(0, rem)],
                    agg_sh.at[pl.ds(base + (RPT // C) * C, rem)])
    pltpu.sync_copy(dzero_v.at[pl.ds(0, rem)],
                    deg_sh.at[pl.ds(base + (RPT // C) * C, rem)])
    plsc.subcore_barrier()

    @pl.loop(0, K)
    def _chunk(j):
        pltpu.async_copy(h2_hbm.at[src_v.at[j]], rows_v, sem).wait()

        @pl.loop(0, C // 16)
        def _scale(c16):
            ovec = ord_v[j, pl.ds(16 * c16, 16)]
            for l in range(16):
                s = ovec[l]
                c = c16 * 16 + l
                for q in range(DH // 16):
                    rows_v[c, pl.ds(16 * q, 16)] = rows_v[c, pl.ds(16 * q, 16)] * s

        pltpu.sync_copy(rows_v, agg_sh.at[dst_v.at[j]], add=True)
        @pl.when(lax.rem(j, 2) == cid)
        def _deg():
            pltpu.sync_copy(dbuf_v, deg_sh.at[dst_v.at[j]], add=True)

    plsc.subcore_barrier()

    pltpu.sync_copy(agg_sh.at[pl.ds(base, RPT)], agg_out.at[cid, pl.ds(base, RPT)])
    pltpu.sync_copy(deg_sh.at[pl.ds(base, RPT)], deg_out.at[cid, pl.ds(base, RPT)])


_sc_call = pl.kernel(
    _sc_body,
    out_type=(
        jax.ShapeDtypeStruct((2, NPAD, DH), jnp.float32),
        jax.ShapeDtypeStruct((2, NPAD, 16), jnp.float32),
    ),
    mesh=plsc.VectorSubcoreMesh(core_axis_name="c", subcore_axis_name="s"),
    compiler_params=pltpu.CompilerParams(use_tc_tiling_on_sc=False),
    scratch_types=[
        pltpu.VMEM((K, C), jnp.int32),
        pltpu.VMEM((K, C), jnp.int32),
        pltpu.VMEM((K, C), jnp.float32),
        pltpu.VMEM((C, DH), jnp.float32),
        pltpu.VMEM((C, 16), jnp.float32),
        pltpu.VMEM((C, 16), jnp.float32),
        pltpu.VMEM_SHARED((NPAD, DH), jnp.float32),
        pltpu.VMEM_SHARED((NPAD, 16), jnp.float32),
        pltpu.SemaphoreType.DMA,
    ],
)


def _tc_body(agg0_ref, agg1_ref, deg0_ref, deg1_ref, w_ref, b_ref, out_ref):
    d = deg0_ref[...] + deg1_ref[...]
    deg = jnp.maximum(d[:, 0:1], 1.0)
    s = jnp.concatenate([agg0_ref[...], agg1_ref[...]], axis=1) / deg
    y = jnp.dot(s, w_ref[...], preferred_element_type=jnp.float32)
    out_ref[...] = y + b_ref[...]


BM = 1000

_tc_call = pl.pallas_call(
    _tc_body,
    out_shape=jax.ShapeDtypeStruct((N, D), jnp.float32),
    grid=(N // BM,),
    in_specs=[
        pl.BlockSpec((BM, DH), lambda i: (i, 0)),
        pl.BlockSpec((BM, DH), lambda i: (i, 0)),
        pl.BlockSpec((BM, 16), lambda i: (i, 0)),
        pl.BlockSpec((BM, 16), lambda i: (i, 0)),
        pl.BlockSpec((D, D), lambda i: (0, 0)),
        pl.BlockSpec((1, D), lambda i: (0, 0)),
    ],
    out_specs=pl.BlockSpec((BM, D), lambda i: (i, 0)),
)


@jax.jit
def kernel(h, edge_index, edge_order, W, b):
    src = edge_index[0]
    dst = edge_index[1]
    pad = EP - E
    srcp = jnp.concatenate([src, jnp.zeros((pad,), jnp.int32)]).reshape(NS, K, C)
    dstp = jnp.concatenate([dst, jnp.full((pad,), N, jnp.int32)]).reshape(NS, K, C)
    ordp = jnp.concatenate(
        [edge_order, jnp.zeros((pad,), jnp.float32)]).reshape(NS, K, C)
    h2 = jnp.concatenate([h[:, :DH], h[:, DH:]], axis=0)

    agg_p, deg_p = _sc_call(h2, srcp, dstp, ordp)
    out = _tc_call(agg_p[0, :N], agg_p[1, :N], deg_p[0, :N], deg_p[1, :N],
                   W, b.reshape(1, D))
    return out


# speedup vs baseline: 4.5976x; 1.2926x over previous
import jax
import jax.numpy as jnp
from jax import lax
from jax.experimental import pallas as pl
from jax.experimental.pallas import tpu as pltpu
from jax.experimental.pallas import tpu_sc as plsc

N = 10000
E = 320000
D = 128
DH = D // 2

NS = 16
C = 128
NB = 4
SB = 32
K = -(-E // (NS * C * SB)) * SB
NBLK = K // SB
EP = NS * C * K
NPAD = 10112
RPT = NPAD // NS
GPB = SB // NB


def _sc_body(h2_hbm, src_hbm, dst_hbm, ord_hbm, agg_out, deg_out,
             sb_v, db_v, ob_v, r0, r1, r2, r3, dbuf_v, dzero_v,
             agg_sh, deg_sh,
             sg0, sg1, sg2, sg3, ss0, ss1, ss2, ss3, sd):
    cid = lax.axis_index("c")
    sid = lax.axis_index("s")
    rows = [r0, r1, r2, r3]
    sg = [sg0, sg1, sg2, sg3]
    ss = [ss0, ss1, ss2, ss3]

    zeros16 = jnp.zeros((16,), jnp.float32)
    lanes = lax.iota(jnp.int32, 16)
    onevec = jnp.where(lanes == 0, 1.0, 0.0).astype(jnp.float32)

    @pl.loop(0, C)
    def _zero_bufs(c):
        for q in range(DH // 16):
            r0[c, pl.ds(16 * q, 16)] = zeros16
        dbuf_v[c, pl.ds(0, 16)] = onevec
        dzero_v[c, pl.ds(0, 16)] = zeros16

    base = sid * RPT
    for k in range(RPT // C):
        pltpu.sync_copy(r0, agg_sh.at[pl.ds(base + C * k, C)])
        pltpu.sync_copy(dzero_v, deg_sh.at[pl.ds(base + C * k, C)])
    rem = RPT % C
    if rem:
        pltpu.sync_copy(r0.at[pl.ds(0, rem)],
                        agg_sh.at[pl.ds(base + (RPT // C) * C, rem)])
        pltpu.sync_copy(dzero_v.at[pl.ds(0, rem)],
                        deg_sh.at[pl.ds(base + (RPT // C) * C, rem)])
    plsc.subcore_barrier()

    @pl.loop(0, NBLK)
    def _blk(blk):
        sl = pl.ds(blk * SB, SB)
        pltpu.sync_copy(src_hbm.at[cid, sid, sl], sb_v)
        pltpu.sync_copy(dst_hbm.at[sid, sl], db_v)
        pltpu.sync_copy(ord_hbm.at[sid, sl], ob_v)

        @pl.loop(0, GPB)
        def _g(g):
            dg = [pltpu.async_copy(h2_hbm.at[sb_v.at[g * NB + b]],
                                   rows[b], sg[b])
                  for b in range(NB)]
            ds_ = []
            for b in range(NB):
                jl = g * NB + b

                @pl.when(cid == 0)
                def _deg():
                    pltpu.sync_copy(dbuf_v, deg_sh.at[db_v.at[jl]], add=True)

                dg[b].wait()

                @pl.loop(0, C // 16)
                def _scale(c16):
                    ovec = ob_v[jl, pl.ds(16 * c16, 16)]
                    for l in range(16):
                        s = ovec[l]
                        c = c16 * 16 + l
                        for w in range(DH // 16):
                            rows[b][c, pl.ds(16 * w, 16)] = (
                                rows[b][c, pl.ds(16 * w, 16)] * s)

                ds_.append(pltpu.async_copy(rows[b], agg_sh.at[db_v.at[jl]],
                                            ss[b], add=True))
            for d in ds_:
                d.wait()

    plsc.subcore_barrier()

    pltpu.sync_copy(agg_sh.at[pl.ds(base, RPT)], agg_out.at[cid, pl.ds(base, RPT)])
    pltpu.sync_copy(deg_sh.at[pl.ds(base, RPT)], deg_out.at[cid, pl.ds(base, RPT)])


_sc_call = pl.kernel(
    _sc_body,
    out_type=(
        jax.ShapeDtypeStruct((2, NPAD, DH), jnp.float32),
        jax.ShapeDtypeStruct((2, NPAD, 16), jnp.float32),
    ),
    mesh=plsc.VectorSubcoreMesh(core_axis_name="c", subcore_axis_name="s"),
    compiler_params=pltpu.CompilerParams(use_tc_tiling_on_sc=False,
                                         needs_layout_passes=False),
    scratch_types=[
        pltpu.VMEM((SB, C), jnp.int32),
        pltpu.VMEM((SB, C), jnp.int32),
        pltpu.VMEM((SB, C), jnp.float32),
        pltpu.VMEM((C, DH), jnp.float32),
        pltpu.VMEM((C, DH), jnp.float32),
        pltpu.VMEM((C, DH), jnp.float32),
        pltpu.VMEM((C, DH), jnp.float32),
        pltpu.VMEM((C, 16), jnp.float32),
        pltpu.VMEM((C, 16), jnp.float32),
        pltpu.VMEM_SHARED((NPAD, DH), jnp.float32),
        pltpu.VMEM_SHARED((NPAD, 16), jnp.float32),
        pltpu.SemaphoreType.DMA,
        pltpu.SemaphoreType.DMA,
        pltpu.SemaphoreType.DMA,
        pltpu.SemaphoreType.DMA,
        pltpu.SemaphoreType.DMA,
        pltpu.SemaphoreType.DMA,
        pltpu.SemaphoreType.DMA,
        pltpu.SemaphoreType.DMA,
        pltpu.SemaphoreType.DMA,
    ],
)


def _tc_body(agg0_ref, agg1_ref, deg_ref, w_ref, b_ref, out_ref):
    deg = jnp.maximum(deg_ref[:, 0:1], 1.0)
    s = jnp.concatenate([agg0_ref[...], agg1_ref[...]], axis=1) / deg
    y = jnp.dot(s, w_ref[...], preferred_element_type=jnp.float32)
    out_ref[...] = y + b_ref[...]


BM = 1000

_tc_call = pl.pallas_call(
    _tc_body,
    out_shape=jax.ShapeDtypeStruct((N, D), jnp.float32),
    grid=(N // BM,),
    in_specs=[
        pl.BlockSpec((BM, DH), lambda i: (i, 0)),
        pl.BlockSpec((BM, DH), lambda i: (i, 0)),
        pl.BlockSpec((BM, 16), lambda i: (i, 0)),
        pl.BlockSpec((D, D), lambda i: (0, 0)),
        pl.BlockSpec((1, D), lambda i: (0, 0)),
    ],
    out_specs=pl.BlockSpec((BM, D), lambda i: (i, 0)),
)


@jax.jit
def kernel(h, edge_index, edge_order, W, b):
    src = edge_index[0]
    dst = edge_index[1]
    pad = EP - E
    srcp = jnp.concatenate([src, jnp.zeros((pad,), jnp.int32)]).reshape(NS, K, C)
    src2 = jnp.stack([srcp, srcp + N])
    dstp = jnp.concatenate([dst, jnp.full((pad,), N, jnp.int32)]).reshape(NS, K, C)
    ordp = jnp.concatenate(
        [edge_order, jnp.zeros((pad,), jnp.float32)]).reshape(NS, K, C)
    h2 = jnp.concatenate([h[:, :DH], h[:, DH:]], axis=0)

    agg_p, deg_p = _sc_call(h2, src2, dstp, ordp)
    out = _tc_call(agg_p[0, :N], agg_p[1, :N], deg_p[0, :N],
                   W, b.reshape(1, D))
    return out


# speedup vs baseline: 5.3371x; 1.1608x over previous
import jax
import jax.numpy as jnp
from jax import lax
from jax.experimental import pallas as pl
from jax.experimental.pallas import tpu as pltpu
from jax.experimental.pallas import tpu_sc as plsc

N = 10000
E = 320000
D = 128
DH = D // 2

NS = 16
C = 128
NB = 8
SB = 16
K = -(-E // (NS * C * SB)) * SB
NBLK = K // SB
EP = NS * C * K
NPAD = 10112
RPT = NPAD // NS
GPB = SB // NB


def _sc_body(h2_hbm, src_hbm, dst_hbm, ord_hbm, agg_out, deg_out,
             sb_v, db_v, ob_v, r0, r1, r2, r3, r4, r5, r6, r7,
             dbuf_v, dzero_v, agg_sh, deg_sh,
             sg0, sg1, sg2, sg3, sg4, sg5, sg6, sg7,
             ss0, ss1, ss2, ss3, ss4, ss5, ss6, ss7, si, sd):
    cid = lax.axis_index("c")
    sid = lax.axis_index("s")
    rows = [r0, r1, r2, r3, r4, r5, r6, r7]
    sg = [sg0, sg1, sg2, sg3, sg4, sg5, sg6, sg7]
    ss = [ss0, ss1, ss2, ss3, ss4, ss5, ss6, ss7]

    zeros16 = jnp.zeros((16,), jnp.float32)
    lanes = lax.iota(jnp.int32, 16)
    onevec = jnp.where(lanes == 0, 1.0, 0.0).astype(jnp.float32)

    @pl.loop(0, C)
    def _zero_bufs(c):
        for q in range(DH // 16):
            r0[c, pl.ds(16 * q, 16)] = zeros16
        dbuf_v[c, pl.ds(0, 16)] = onevec
        dzero_v[c, pl.ds(0, 16)] = zeros16

    base = sid * RPT
    for k in range(RPT // C):
        pltpu.sync_copy(r0, agg_sh.at[pl.ds(base + C * k, C)])
        pltpu.sync_copy(dzero_v, deg_sh.at[pl.ds(base + C * k, C)])
    rem = RPT % C
    if rem:
        pltpu.sync_copy(r0.at[pl.ds(0, rem)],
                        agg_sh.at[pl.ds(base + (RPT // C) * C, rem)])
        pltpu.sync_copy(dzero_v.at[pl.ds(0, rem)],
                        deg_sh.at[pl.ds(base + (RPT // C) * C, rem)])
    plsc.subcore_barrier()

    @pl.loop(0, NBLK)
    def _blk(blk):
        sl = pl.ds(blk * SB, SB)
        d1 = pltpu.async_copy(src_hbm.at[cid, sid, sl], sb_v, si)
        d2 = pltpu.async_copy(dst_hbm.at[sid, sl], db_v, si)
        d3 = pltpu.async_copy(ord_hbm.at[sid, sl], ob_v, si)
        d1.wait()
        d2.wait()
        d3.wait()

        @pl.loop(0, GPB)
        def _g(g):
            dg = [pltpu.async_copy(h2_hbm.at[sb_v.at[g * NB + b]],
                                   rows[b], sg[b])
                  for b in range(NB)]
            ds_ = []
            dd_ = []
            for b in range(NB):
                jl = g * NB + b

                ddesc = pltpu.make_async_copy(
                    dbuf_v, deg_sh.at[db_v.at[jl]], sd)
                dd_.append(ddesc)

                @pl.when(cid == 0)
                def _deg():
                    ddesc.start(add=True)

                dg[b].wait()

                @pl.loop(0, C // 16)
                def _scale(c16):
                    ovec = ob_v[jl, pl.ds(16 * c16, 16)]
                    for l in range(16):
                        s = ovec[l]
                        c = c16 * 16 + l
                        for w in range(DH // 16):
                            rows[b][c, pl.ds(16 * w, 16)] = (
                                rows[b][c, pl.ds(16 * w, 16)] * s)

                ds_.append(pltpu.async_copy(rows[b], agg_sh.at[db_v.at[jl]],
                                            ss[b], add=True))
            for d in ds_:
                d.wait()

            @pl.when(cid == 0)
            def _dd():
                for d in dd_:
                    d.wait()

    plsc.subcore_barrier()

    pltpu.sync_copy(agg_sh.at[pl.ds(base, RPT)], agg_out.at[cid, pl.ds(base, RPT)])
    pltpu.sync_copy(deg_sh.at[pl.ds(base, RPT)], deg_out.at[cid, pl.ds(base, RPT)])


_sc_call = pl.kernel(
    _sc_body,
    out_type=(
        jax.ShapeDtypeStruct((2, NPAD, DH), jnp.float32),
        jax.ShapeDtypeStruct((2, NPAD, 16), jnp.float32),
    ),
    mesh=plsc.VectorSubcoreMesh(core_axis_name="c", subcore_axis_name="s"),
    compiler_params=pltpu.CompilerParams(use_tc_tiling_on_sc=False,
                                         needs_layout_passes=False),
    scratch_types=(
        [
            pltpu.VMEM((SB, C), jnp.int32),
            pltpu.VMEM((SB, C), jnp.int32),
            pltpu.VMEM((SB, C), jnp.float32),
        ]
        + [pltpu.VMEM((C, DH), jnp.float32)] * 8
        + [
            pltpu.VMEM((C, 16), jnp.float32),
            pltpu.VMEM((C, 16), jnp.float32),
            pltpu.VMEM_SHARED((NPAD, DH), jnp.float32),
            pltpu.VMEM_SHARED((NPAD, 16), jnp.float32),
        ]
        + [pltpu.SemaphoreType.DMA] * 18
    ),
)


def _tc_body(agg0_ref, agg1_ref, deg_ref, w_ref, b_ref, out_ref):
    deg = jnp.maximum(deg_ref[:, 0:1], 1.0)
    s = jnp.concatenate([agg0_ref[...], agg1_ref[...]], axis=1) / deg
    y = jnp.dot(s, w_ref[...], preferred_element_type=jnp.float32)
    out_ref[...] = y + b_ref[...]


BM = 1000

_tc_call = pl.pallas_call(
    _tc_body,
    out_shape=jax.ShapeDtypeStruct((N, D), jnp.float32),
    grid=(N // BM,),
    in_specs=[
        pl.BlockSpec((BM, DH), lambda i: (i, 0)),
        pl.BlockSpec((BM, DH), lambda i: (i, 0)),
        pl.BlockSpec((BM, 16), lambda i: (i, 0)),
        pl.BlockSpec((D, D), lambda i: (0, 0)),
        pl.BlockSpec((1, D), lambda i: (0, 0)),
    ],
    out_specs=pl.BlockSpec((BM, D), lambda i: (i, 0)),
)


@jax.jit
def kernel(h, edge_index, edge_order, W, b):
    src = edge_index[0]
    dst = edge_index[1]
    pad = EP - E
    srcp = jnp.concatenate([src, jnp.zeros((pad,), jnp.int32)]).reshape(NS, K, C)
    src2 = jnp.stack([srcp, srcp + N])
    dstp = jnp.concatenate([dst, jnp.full((pad,), N, jnp.int32)]).reshape(NS, K, C)
    ordp = jnp.concatenate(
        [edge_order, jnp.zeros((pad,), jnp.float32)]).reshape(NS, K, C)
    h2 = jnp.concatenate([h[:, :DH], h[:, DH:]], axis=0)

    agg_p, deg_p = _sc_call(h2, src2, dstp, ordp)
    out = _tc_call(agg_p[0, :N], agg_p[1, :N], deg_p[0, :N],
                   W, b.reshape(1, D))
    return out


# speedup vs baseline: 5.5856x; 1.0466x over previous
import jax
import jax.numpy as jnp
from jax import lax
from jax.experimental import pallas as pl
from jax.experimental.pallas import tpu as pltpu
from jax.experimental.pallas import tpu_sc as plsc

N = 10000
E = 320000
D = 128
DH = D // 2

NS = 16
C = 128
NB = 8
SB = 16
K = -(-E // (NS * C * SB)) * SB
NBLK = K // SB
EP = NS * C * K
NPAD = 10112
RPT = NPAD // NS
GPB = SB // NB


def _sc_body(h2_hbm, src_hbm, dst_hbm, ord_hbm, agg_out, deg_out,
             sb_v, db_v, ob_v, r0, r1, r2, r3, r4, r5, r6, r7,
             dbuf_v, dzero_v, agg_sh, deg_sh,
             sg0, sg1, sg2, sg3, sg4, sg5, sg6, sg7,
             ss0, ss1, ss2, ss3, ss4, ss5, ss6, ss7, si, sd):
    cid = lax.axis_index("c")
    sid = lax.axis_index("s")
    rows = [r0, r1, r2, r3, r4, r5, r6, r7]
    sg = [sg0, sg1, sg2, sg3, sg4, sg5, sg6, sg7]
    ss = [ss0, ss1, ss2, ss3, ss4, ss5, ss6, ss7]

    zeros16 = jnp.zeros((16,), jnp.float32)
    lanes = lax.iota(jnp.int32, 16)
    onevec = jnp.where(lanes == 0, 1.0, 0.0).astype(jnp.float32)

    @pl.loop(0, C)
    def _zero_bufs(c):
        for q in range(DH // 16):
            r0[c, pl.ds(16 * q, 16)] = zeros16
        dbuf_v[c, pl.ds(0, 16)] = onevec
        dzero_v[c, pl.ds(0, 16)] = zeros16

    base = sid * RPT
    for k in range(RPT // C):
        pltpu.sync_copy(r0, agg_sh.at[pl.ds(base + C * k, C)])
        pltpu.sync_copy(dzero_v, deg_sh.at[pl.ds(base + C * k, C)])
    rem = RPT % C
    if rem:
        pltpu.sync_copy(r0.at[pl.ds(0, rem)],
                        agg_sh.at[pl.ds(base + (RPT // C) * C, rem)])
        pltpu.sync_copy(dzero_v.at[pl.ds(0, rem)],
                        deg_sh.at[pl.ds(base + (RPT // C) * C, rem)])
    plsc.subcore_barrier()

    @pl.loop(0, NBLK)
    def _blk(blk):
        sl = pl.ds(blk * SB, SB)
        d1 = pltpu.async_copy(src_hbm.at[cid, sid, sl], sb_v, si)
        d2 = pltpu.async_copy(dst_hbm.at[sid, sl], db_v, si)
        d3 = pltpu.async_copy(ord_hbm.at[sid, sl], ob_v, si)
        d1.wait()
        d2.wait()
        d3.wait()

        @pl.loop(0, GPB)
        def _g(g):
            dg = [pltpu.async_copy(h2_hbm.at[sb_v.at[g * NB + b]],
                                   rows[b], sg[b])
                  for b in range(NB)]
            ds_ = []
            dd_ = []
            for b in range(NB):
                jl = g * NB + b

                ddesc = pltpu.make_async_copy(
                    dbuf_v, deg_sh.at[db_v.at[jl]], sd)
                dd_.append(ddesc)

                @pl.when(cid == 0)
                def _deg():
                    ddesc.start(add=True)

                dg[b].wait()

                @plsc.parallel_loop(0, C // 16, unroll=2)
                def _scale(c16):
                    ovec = ob_v[jl, pl.ds(16 * c16, 16)]
                    for l in range(16):
                        s = ovec[l]
                        c = c16 * 16 + l
                        for w in range(DH // 16):
                            rows[b][c, pl.ds(16 * w, 16)] = (
                                rows[b][c, pl.ds(16 * w, 16)] * s)

                ds_.append(pltpu.async_copy(rows[b], agg_sh.at[db_v.at[jl]],
                                            ss[b], add=True))
            for d in ds_:
                d.wait()

            @pl.when(cid == 0)
            def _dd():
                for d in dd_:
                    d.wait()

    plsc.subcore_barrier()

    pltpu.sync_copy(agg_sh.at[pl.ds(base, RPT)], agg_out.at[cid, pl.ds(base, RPT)])
    pltpu.sync_copy(deg_sh.at[pl.ds(base, RPT)], deg_out.at[cid, pl.ds(base, RPT)])


_sc_call = pl.kernel(
    _sc_body,
    out_type=(
        jax.ShapeDtypeStruct((2, NPAD, DH), jnp.float32),
        jax.ShapeDtypeStruct((2, NPAD, 16), jnp.float32),
    ),
    mesh=plsc.VectorSubcoreMesh(core_axis_name="c", subcore_axis_name="s"),
    compiler_params=pltpu.CompilerParams(use_tc_tiling_on_sc=False,
                                         needs_layout_passes=False),
    scratch_types=(
        [
            pltpu.VMEM((SB, C), jnp.int32),
            pltpu.VMEM((SB, C), jnp.int32),
            pltpu.VMEM((SB, C), jnp.float32),
        ]
        + [pltpu.VMEM((C, DH), jnp.float32)] * 8
        + [
            pltpu.VMEM((C, 16), jnp.float32),
            pltpu.VMEM((C, 16), jnp.float32),
            pltpu.VMEM_SHARED((NPAD, DH), jnp.float32),
            pltpu.VMEM_SHARED((NPAD, 16), jnp.float32),
        ]
        + [pltpu.SemaphoreType.DMA] * 18
    ),
)


def _tc_body(agg0_ref, agg1_ref, deg_ref, w_ref, b_ref, out_ref):
    deg = jnp.maximum(deg_ref[:, 0:1], 1.0)
    s = jnp.concatenate([agg0_ref[...], agg1_ref[...]], axis=1) / deg
    y = jnp.dot(s, w_ref[...], preferred_element_type=jnp.float32)
    out_ref[...] = y + b_ref[...]


BM = 1000

_tc_call = pl.pallas_call(
    _tc_body,
    out_shape=jax.ShapeDtypeStruct((N, D), jnp.float32),
    grid=(N // BM,),
    in_specs=[
        pl.BlockSpec((BM, DH), lambda i: (i, 0)),
        pl.BlockSpec((BM, DH), lambda i: (i, 0)),
        pl.BlockSpec((BM, 16), lambda i: (i, 0)),
        pl.BlockSpec((D, D), lambda i: (0, 0)),
        pl.BlockSpec((1, D), lambda i: (0, 0)),
    ],
    out_specs=pl.BlockSpec((BM, D), lambda i: (i, 0)),
)


@jax.jit
def kernel(h, edge_index, edge_order, W, b):
    src = edge_index[0]
    dst = edge_index[1]
    pad = EP - E
    srcp = jnp.concatenate([src, jnp.zeros((pad,), jnp.int32)]).reshape(NS, K, C)
    src2 = jnp.stack([srcp, srcp + N])
    dstp = jnp.concatenate([dst, jnp.full((pad,), N, jnp.int32)]).reshape(NS, K, C)
    ordp = jnp.concatenate(
        [edge_order, jnp.zeros((pad,), jnp.float32)]).reshape(NS, K, C)
    h2 = jnp.concatenate([h[:, :DH], h[:, DH:]], axis=0)

    agg_p, deg_p = _sc_call(h2, src2, dstp, ordp)
    out = _tc_call(agg_p[0, :N], agg_p[1, :N], deg_p[0, :N],
                   W, b.reshape(1, D))
    return out
